# baseline (speedup 1.0000x reference)
import functools

import jax
import jax.numpy as jnp
from jax import lax
from jax.experimental import pallas as pl
from jax.experimental.pallas import tpu as pltpu
from jax.experimental.pallas import tpu_sc as plsc

B = 16384
OUT_D = 304

_FEATS = [
    (2, 0, 16), (3, 1, 16), (4, 2, 16), (5, 3, 32), (6, 4, 32),
    (7, 3, 32), (8, 4, 32), (9, 5, 16), (10, 6, 16), (11, 7, 16),
    (12, 8, 16), (13, 9, 16), (14, 10, 8), (15, 11, 8), (16, 12, 8),
    (17, 13, 8), (18, 14, 8), (19, 15, 8),
]
_OFFS = []
_acc = 0
for _, _, _w in _FEATS:
    _OFFS.append(_acc)
    _acc += _w
assert _acc == OUT_D

_NC, _NS = 2, 16
_NW = _NC * _NS
_BPW = B // _NW


def _make_sc_gather():
    mesh = plsc.VectorSubcoreMesh(core_axis_name="c", subcore_axis_name="s")
    scratch = [
        pltpu.VMEM((_BPW,), jnp.int32),
        pltpu.VMEM((_BPW, 8), jnp.float32),
        pltpu.VMEM((_BPW, 16), jnp.float32),
        pltpu.VMEM((_BPW, 32), jnp.float32),
        pltpu.SemaphoreType.DMA,
    ]

    @functools.partial(
        pl.kernel,
        out_type=jax.ShapeDtypeStruct((B, OUT_D), jnp.float32),
        mesh=mesh,
        scratch_types=scratch,
        compiler_params=pltpu.CompilerParams(use_tc_tiling_on_sc=False),
    )
    def sc_gather(idx_hbm, *rest):
        tables = rest[:16]
        out_hbm = rest[16]
        idx_v, buf8, buf16, buf32 = rest[17:21]
        sem = rest[21]
        bufs = {8: buf8, 16: buf16, 32: buf32}

        wid = lax.axis_index("s") * _NC + lax.axis_index("c")
        base = wid * _BPW
        for f, (_, slot, w) in enumerate(_FEATS):
            pltpu.sync_copy(idx_hbm.at[pl.ds(f * B + base, _BPW)], idx_v)
            buf = bufs[w]
            pltpu.async_copy(tables[slot].at[idx_v], buf, sem).wait()
            pltpu.sync_copy(
                buf, out_hbm.at[pl.ds(base, _BPW), pl.ds(_OFFS[f], w)])

    return sc_gather


_sc_gather = _make_sc_gather()


def kernel(inputs, W_highway, W_length, W_radian, W_lon, W_lat, W_lanes,
           W_c_centrality, W_b_centrality, W_h_centrality, W_degree,
           W_cultural, W_education, W_food, W_health, W_service,
           W_transportation):
    idx = jnp.transpose(inputs[:, 2:20].astype(jnp.int32)).reshape(-1)
    tables = (W_highway, W_length, W_radian, W_lon, W_lat, W_lanes,
              W_c_centrality, W_b_centrality, W_h_centrality, W_degree,
              W_cultural, W_education, W_food, W_health, W_service,
              W_transportation)
    return _sc_gather(idx, *tables)


# speedup vs baseline: 1.0261x; 1.0261x over previous
import functools

import jax
import jax.numpy as jnp
from jax import lax
from jax.experimental import pallas as pl
from jax.experimental.pallas import tpu as pltpu
from jax.experimental.pallas import tpu_sc as plsc

B = 16384
OUT_D = 304

_FEATS = [
    (2, 0, 16), (3, 1, 16), (4, 2, 16), (5, 3, 32), (6, 4, 32),
    (7, 3, 32), (8, 4, 32), (9, 5, 16), (10, 6, 16), (11, 7, 16),
    (12, 8, 16), (13, 9, 16), (14, 10, 8), (15, 11, 8), (16, 12, 8),
    (17, 13, 8), (18, 14, 8), (19, 15, 8),
]
_NF = len(_FEATS)
_OFFS = []
_acc = 0
for _, _, _w in _FEATS:
    _OFFS.append(_acc)
    _acc += _w
assert _acc == OUT_D

_NC, _NS = 2, 16
_NW = _NC * _NS
_BPW = B // _NW
_CH = 128
_NCHUNK = _BPW // _CH


def _make_sc_gather():
    mesh = plsc.VectorSubcoreMesh(core_axis_name="c", subcore_axis_name="s")
    scratch = [pltpu.VMEM((_NF, _BPW), jnp.int32)]
    for _p in range(2):
        scratch += [pltpu.VMEM((_CH, w), jnp.float32) for _, _, w in _FEATS]
    scratch += [pltpu.SemaphoreType.DMA] * 4

    @functools.partial(
        pl.kernel,
        out_type=jax.ShapeDtypeStruct((B, OUT_D), jnp.float32),
        mesh=mesh,
        scratch_types=scratch,
        compiler_params=pltpu.CompilerParams(use_tc_tiling_on_sc=False),
    )
    def sc_gather(idx_hbm, *rest):
        tables = rest[:16]
        out_hbm = rest[16]
        idx_v = rest[17]
        bufs = [rest[18:18 + _NF], rest[18 + _NF:18 + 2 * _NF]]
        semg = rest[18 + 2 * _NF:20 + 2 * _NF]
        semw = rest[20 + 2 * _NF:22 + 2 * _NF]

        wid = lax.axis_index("s") * _NC + lax.axis_index("c")
        base = wid * _BPW
        pltpu.sync_copy(idx_hbm.at[wid], idx_v)

        wr = [[], []]
        for k in range(_NCHUNK):
            p = k % 2
            for d in wr[p]:
                d.wait()
            wr[p] = []
            descs = []
            for f, (_, slot, w) in enumerate(_FEATS):
                descs.append(pltpu.async_copy(
                    tables[slot].at[idx_v.at[f, pl.ds(k * _CH, _CH)]],
                    bufs[p][f], semg[p]))
            for d in descs:
                d.wait()
            row0 = base + k * _CH
            for f, (_, slot, w) in enumerate(_FEATS):
                wr[p].append(pltpu.async_copy(
                    bufs[p][f],
                    out_hbm.at[pl.ds(row0, _CH), pl.ds(_OFFS[f], w)],
                    semw[p]))
        for p in range(2):
            for d in wr[p]:
                d.wait()

    return sc_gather


_sc_gather = _make_sc_gather()


def kernel(inputs, W_highway, W_length, W_radian, W_lon, W_lat, W_lanes,
           W_c_centrality, W_b_centrality, W_h_centrality, W_degree,
           W_cultural, W_education, W_food, W_health, W_service,
           W_transportation):
    idx = (jnp.transpose(inputs[:, 2:20].astype(jnp.int32))
           .reshape(_NF, _NW, _BPW)
           .transpose(1, 0, 2))
    tables = (W_highway, W_length, W_radian, W_lon, W_lat, W_lanes,
              W_c_centrality, W_b_centrality, W_h_centrality, W_degree,
              W_cultural, W_education, W_food, W_health, W_service,
              W_transportation)
    return _sc_gather(idx, *tables)


# speedup vs baseline: 1.2146x; 1.1837x over previous
import functools

import jax
import jax.numpy as jnp
from jax import lax
from jax.experimental import pallas as pl
from jax.experimental.pallas import tpu as pltpu
from jax.experimental.pallas import tpu_sc as plsc

B = 16384
V = 100000
OUT_D = 304
_HALF = B // 2
_GVECS = _HALF // 16
_CHUNK = 4096
_NCHUNK = 24
_REM_OFF = _NCHUNK * _CHUNK
_REM = 99968 - _REM_OFF
_TAIL_OFF, _TAIL = 99968, V - 99968

_FEATS = [(0, 16), (1, 16), (2, 16), (3, 32), (4, 32), (3, 32), (4, 32),
          (5, 16), (6, 16), (7, 16), (8, 16), (9, 16), (10, 8), (11, 8),
          (12, 8), (13, 8), (14, 8), (15, 8)]
_OFFS = []
_acc = 0
for _, _w in _FEATS:
    _OFFS.append(_acc)
    _acc += _w
assert _acc == OUT_D

_BANDS = []
for _f, (_slot, _w) in enumerate(_FEATS):
    if _f in (5, 6):
        continue
    for _t in range(_w // 8):
        dual = _f + 2 if _f in (3, 4) else None
        _BANDS.append((
            _slot, 8 * _t, _f, _OFFS[_f] + 8 * _t,
            dual, (_OFFS[_f + 2] + 8 * _t) if dual is not None else 0,
        ))
assert len(_BANDS) == 30


def _make_sc_gather():
    mesh = plsc.VectorSubcoreMesh(core_axis_name="c", subcore_axis_name="s")
    scratch = [
        pltpu.VMEM_SHARED((8, _CHUNK), jnp.float32),
        pltpu.VMEM_SHARED((8, _CHUNK), jnp.float32),
        pltpu.VMEM_SHARED((8, B // 4), jnp.float32),
        pltpu.VMEM_SHARED((8, _REM), jnp.float32),
        pltpu.VMEM((V,), jnp.float32),
        pltpu.VMEM((8, _TAIL), jnp.float32),
        pltpu.VMEM((_HALF,), jnp.int32),
        pltpu.VMEM((_HALF,), jnp.float32),
        pltpu.SemaphoreType.DMA,
    ]

    @functools.partial(
        pl.kernel,
        out_type=jax.ShapeDtypeStruct((OUT_D, B), jnp.float32),
        mesh=mesh,
        scratch_types=scratch,
        compiler_params=pltpu.CompilerParams(needs_layout_passes=False),
    )
    def sc_gather(idx_hbm, *rest):
        tables = rest[:16]
        tails = rest[16:32]
        out_hbm = rest[32]
        (chunk_a, chunk_b, out_sp, rem_sp, row_v, tail_v, idx_v, g_v,
         sem) = rest[33:42]
        chunk_bufs = (chunk_a, chunk_b)

        cid = lax.axis_index("c")
        sid = lax.axis_index("s")
        c = lax.rem(sid, 8)
        h = sid // 8

        def gather_half(fpos):
            off = pl.multiple_of(fpos * B + h * _HALF, 8)
            pltpu.sync_copy(idx_hbm.at[pl.ds(off, _HALF)], idx_v)

            def body(j, _):
                iv = idx_v[pl.ds(j * 16, 16)]
                g_v[pl.ds(j * 16, 16)] = plsc.load_gather(row_v, [iv])
                return ()

            lax.fori_loop(0, _GVECS, body, (), unroll=8)

        def write_band(out_base):
            q = B // 4
            for hh in range(4):
                @pl.when(h == hh // 2)
                def _():
                    pltpu.sync_copy(g_v.at[pl.ds((hh % 2) * q, q)],
                                    out_sp.at[c])
                plsc.subcore_barrier()
                cw = q // 16
                co = pl.multiple_of(sid * cw, 128)
                pltpu.sync_copy(
                    out_sp.at[:, pl.ds(co, cw)],
                    out_hbm.at[pl.ds(out_base, 8),
                               pl.ds(hh * q + co, cw)])
                plsc.subcore_barrier()

        for i, (slot, trow, fpos, obase, dual, dbase) in enumerate(_BANDS):
            @pl.when(cid == (i % 2))
            def _():
                @pl.when(sid == 15)
                def _():
                    pltpu.sync_copy(
                        tables[slot].at[pl.ds(trow, 8), pl.ds(_REM_OFF, _REM)],
                        rem_sp)
                pltpu.sync_copy(tails[slot].at[pl.ds(trow, 8), :], tail_v)
                def chunk_body(m, _):
                    for p in range(2):
                        k = m * 2 + p
                        off = pl.multiple_of(k * _CHUNK, 128)

                        @pl.when(sid == lax.rem(k, 16))
                        def _():
                            pltpu.sync_copy(
                                tables[slot].at[pl.ds(trow, 8),
                                                pl.ds(off, _CHUNK)],
                                chunk_bufs[p])
                        plsc.subcore_barrier()
                        pltpu.sync_copy(
                            chunk_bufs[p].at[c],
                            row_v.at[pl.ds(off, _CHUNK)])
                    return ()

                lax.fori_loop(0, _NCHUNK // 2, chunk_body, ())
                plsc.subcore_barrier()
                pltpu.sync_copy(rem_sp.at[c],
                                row_v.at[pl.ds(_REM_OFF, _REM)])
                for j in range(_TAIL // 16):
                    row_v[pl.ds(_TAIL_OFF + j * 16, 16)] = (
                        tail_v[c, pl.ds(j * 16, 16)])
                gather_half(fpos)
                write_band(obase)
                if dual is not None:
                    gather_half(dual)
                    write_band(dbase)

    return sc_gather


_sc_gather = _make_sc_gather()


def kernel(inputs, W_highway, W_length, W_radian, W_lon, W_lat, W_lanes,
           W_c_centrality, W_b_centrality, W_h_centrality, W_degree,
           W_cultural, W_education, W_food, W_health, W_service,
           W_transportation):
    idx = jnp.transpose(inputs[:, 2:20].astype(jnp.int32)).reshape(-1)
    tables = (W_highway, W_length, W_radian, W_lon, W_lat, W_lanes,
              W_c_centrality, W_b_centrality, W_h_centrality, W_degree,
              W_cultural, W_education, W_food, W_health, W_service,
              W_transportation)
    tt = tuple(t.T for t in tables)
    tails = tuple(t[:, _TAIL_OFF:] for t in tt)
    out_t = _sc_gather(idx, *tt, *tails)
    return jnp.transpose(out_t)


# speedup vs baseline: 1.6270x; 1.3396x over previous
import functools

import jax
import jax.numpy as jnp
from jax import lax
from jax.experimental import pallas as pl
from jax.experimental.pallas import tpu as pltpu
from jax.experimental.pallas import tpu_sc as plsc

B = 16384
V = 100000
OUT_D = 304
_HALF = B // 2
_GVECS = _HALF // 16
_CHUNK = 8192
_NCHUNK = 12
_REM_OFF = _NCHUNK * _CHUNK
_REM = 99968 - _REM_OFF
_TAIL_OFF, _TAIL = 99968, V - 99968

_FEATS = [(0, 16), (1, 16), (2, 16), (3, 32), (4, 32), (3, 32), (4, 32),
          (5, 16), (6, 16), (7, 16), (8, 16), (9, 16), (10, 8), (11, 8),
          (12, 8), (13, 8), (14, 8), (15, 8)]
_OFFS = []
_acc = 0
for _, _w in _FEATS:
    _OFFS.append(_acc)
    _acc += _w
assert _acc == OUT_D

_BANDS = []
for _f, (_slot, _w) in enumerate(_FEATS):
    if _f in (5, 6):
        continue
    for _t in range(_w // 8):
        dual = _f + 2 if _f in (3, 4) else None
        _BANDS.append((
            _slot, 8 * _t, _f, _OFFS[_f] + 8 * _t,
            dual, (_OFFS[_f + 2] + 8 * _t) if dual is not None else 0,
        ))
assert len(_BANDS) == 30


def _make_sc_gather():
    mesh = plsc.VectorSubcoreMesh(core_axis_name="c", subcore_axis_name="s")
    scratch = [
        pltpu.VMEM_SHARED((8, _CHUNK), jnp.float32),
        pltpu.VMEM_SHARED((8, _CHUNK), jnp.float32),
        pltpu.VMEM_SHARED((8, B // 4), jnp.float32),
        pltpu.VMEM_SHARED((8, _REM), jnp.float32),
        pltpu.VMEM((V,), jnp.float32),
        pltpu.VMEM((8, _TAIL), jnp.float32),
        pltpu.VMEM((_HALF,), jnp.int32),
        pltpu.VMEM((_HALF,), jnp.float32),
        pltpu.SemaphoreType.DMA,
    ]

    @functools.partial(
        pl.kernel,
        out_type=jax.ShapeDtypeStruct((OUT_D, B), jnp.float32),
        mesh=mesh,
        scratch_types=scratch,
        compiler_params=pltpu.CompilerParams(needs_layout_passes=False),
    )
    def sc_gather(idx_hbm, *rest):
        tables = rest[:16]
        tails = rest[16:32]
        out_hbm = rest[32]
        (chunk_a, chunk_b, out_sp, rem_sp, row_v, tail_v, idx_v, g_v,
         sem) = rest[33:42]
        chunk_bufs = (chunk_a, chunk_b)

        cid = lax.axis_index("c")
        sid = lax.axis_index("s")
        c = lax.rem(sid, 8)
        h = sid // 8

        def gather_half(fpos):
            off = pl.multiple_of(fpos * B + h * _HALF, 8)
            pltpu.sync_copy(idx_hbm.at[pl.ds(off, _HALF)], idx_v)

            def body(j, _):
                iv = idx_v[pl.ds(j * 16, 16)]
                g_v[pl.ds(j * 16, 16)] = plsc.load_gather(row_v, [iv])
                return ()

            lax.fori_loop(0, _GVECS, body, (), unroll=8)

        def write_band(out_base):
            q = B // 4
            for hh in range(4):
                @pl.when(h == hh // 2)
                def _():
                    pltpu.sync_copy(g_v.at[pl.ds((hh % 2) * q, q)],
                                    out_sp.at[c])
                plsc.subcore_barrier()
                cw = q // 16
                co = pl.multiple_of(sid * cw, 128)
                pltpu.sync_copy(
                    out_sp.at[:, pl.ds(co, cw)],
                    out_hbm.at[pl.ds(out_base, 8),
                               pl.ds(hh * q + co, cw)])
                plsc.subcore_barrier()

        for i, (slot, trow, fpos, obase, dual, dbase) in enumerate(_BANDS):
            @pl.when(cid == (i % 2))
            def _():
                @pl.when(sid == 15)
                def _():
                    pltpu.sync_copy(
                        tables[slot].at[pl.ds(trow, 8), pl.ds(_REM_OFF, _REM)],
                        rem_sp)
                pltpu.sync_copy(tails[slot].at[pl.ds(trow, 8), :], tail_v)
                def chunk_body(m, _):
                    for p in range(2):
                        k = m * 2 + p
                        off = pl.multiple_of(k * _CHUNK, 128)

                        @pl.when(sid == lax.rem(k, 16))
                        def _():
                            pltpu.sync_copy(
                                tables[slot].at[pl.ds(trow, 8),
                                                pl.ds(off, _CHUNK)],
                                chunk_bufs[p])
                        plsc.subcore_barrier()
                        pltpu.sync_copy(
                            chunk_bufs[p].at[c],
                            row_v.at[pl.ds(off, _CHUNK)])
                    return ()

                lax.fori_loop(0, _NCHUNK // 2, chunk_body, ())
                plsc.subcore_barrier()
                pltpu.sync_copy(rem_sp.at[c],
                                row_v.at[pl.ds(_REM_OFF, _REM)])
                for j in range(_TAIL // 16):
                    row_v[pl.ds(_TAIL_OFF + j * 16, 16)] = (
                        tail_v[c, pl.ds(j * 16, 16)])
                gather_half(fpos)
                write_band(obase)
                if dual is not None:
                    gather_half(dual)
                    write_band(dbase)

    return sc_gather


_sc_gather = _make_sc_gather()


def kernel(inputs, W_highway, W_length, W_radian, W_lon, W_lat, W_lanes,
           W_c_centrality, W_b_centrality, W_h_centrality, W_degree,
           W_cultural, W_education, W_food, W_health, W_service,
           W_transportation):
    idx = jnp.transpose(inputs[:, 2:20].astype(jnp.int32)).reshape(-1)
    tables = (W_highway, W_length, W_radian, W_lon, W_lat, W_lanes,
              W_c_centrality, W_b_centrality, W_h_centrality, W_degree,
              W_cultural, W_education, W_food, W_health, W_service,
              W_transportation)
    tt = tuple(t.T for t in tables)
    tails = tuple(t[:, _TAIL_OFF:] for t in tt)
    out_t = _sc_gather(idx, *tt, *tails)
    return jnp.transpose(out_t)


# speedup vs baseline: 1.7254x; 1.0605x over previous
import functools

import jax
import jax.numpy as jnp
from jax import lax
from jax.experimental import pallas as pl
from jax.experimental.pallas import tpu as pltpu
from jax.experimental.pallas import tpu_sc as plsc

B = 16384
V = 100000
OUT_D = 304
_HALF = B // 2
_GVECS = _HALF // 16
_CHUNK = 8192
_NCHUNK = 12
_REM_OFF = _NCHUNK * _CHUNK
_REM = 99968 - _REM_OFF
_TAIL_OFF, _TAIL = 99968, V - 99968

_FEATS = [(0, 16), (1, 16), (2, 16), (3, 32), (4, 32), (3, 32), (4, 32),
          (5, 16), (6, 16), (7, 16), (8, 16), (9, 16), (10, 8), (11, 8),
          (12, 8), (13, 8), (14, 8), (15, 8)]
_OFFS = []
_acc = 0
for _, _w in _FEATS:
    _OFFS.append(_acc)
    _acc += _w
assert _acc == OUT_D

_BANDS = []
for _f, (_slot, _w) in enumerate(_FEATS):
    if _f in (5, 6):
        continue
    for _t in range(_w // 8):
        dual = _f + 2 if _f in (3, 4) else None
        _BANDS.append((
            _slot, 8 * _t, _f, _OFFS[_f] + 8 * _t,
            dual, (_OFFS[_f + 2] + 8 * _t) if dual is not None else 0,
        ))
assert len(_BANDS) == 30


def _make_sc_gather():
    mesh = plsc.VectorSubcoreMesh(core_axis_name="c", subcore_axis_name="s")
    scratch = [
        pltpu.VMEM_SHARED((8, _CHUNK), jnp.float32),
        pltpu.VMEM_SHARED((8, _CHUNK), jnp.float32),
        pltpu.VMEM_SHARED((8, _HALF), jnp.float32),
        pltpu.VMEM_SHARED((8, _REM), jnp.float32),
        pltpu.VMEM((V,), jnp.float32),
        pltpu.VMEM((8, _TAIL), jnp.float32),
        pltpu.VMEM((_HALF,), jnp.int32),
        pltpu.VMEM((_HALF,), jnp.float32),
        pltpu.SemaphoreType.DMA,
    ]

    @functools.partial(
        pl.kernel,
        out_type=jax.ShapeDtypeStruct((OUT_D, B), jnp.float32),
        mesh=mesh,
        scratch_types=scratch,
        compiler_params=pltpu.CompilerParams(needs_layout_passes=False),
    )
    def sc_gather(idx_hbm, *rest):
        tables = rest[:16]
        tails = rest[16:32]
        out_hbm = rest[32]
        (chunk_a, chunk_b, out_sp, rem_sp, row_v, tail_v, idx_v, g_v,
         sem) = rest[33:42]
        chunk_bufs = (chunk_a, chunk_b)

        cid = lax.axis_index("c")
        sid = lax.axis_index("s")
        c = lax.rem(sid, 8)
        h = sid // 8

        def gather_half(fpos):
            off = pl.multiple_of(fpos * B + h * _HALF, 8)
            pltpu.sync_copy(idx_hbm.at[pl.ds(off, _HALF)], idx_v)

            def body(j, _):
                iv = idx_v[pl.ds(j * 16, 16)]
                g_v[pl.ds(j * 16, 16)] = plsc.load_gather(row_v, [iv])
                return ()

            lax.fori_loop(0, _GVECS, body, (), unroll=8)

        def write_band(out_base):
            for hh in range(2):
                @pl.when(h == hh)
                def _():
                    pltpu.sync_copy(g_v, out_sp.at[c])
                plsc.subcore_barrier()
                cw = _HALF // 16
                co = pl.multiple_of(sid * cw, 128)
                pltpu.sync_copy(
                    out_sp.at[:, pl.ds(co, cw)],
                    out_hbm.at[pl.ds(out_base, 8),
                               pl.ds(hh * _HALF + co, cw)])
                plsc.subcore_barrier()

        for i, (slot, trow, fpos, obase, dual, dbase) in enumerate(_BANDS):
            @pl.when(cid == (i % 2))
            def _():
                @pl.when(sid == 15)
                def _():
                    pltpu.sync_copy(
                        tables[slot].at[pl.ds(trow, 8), pl.ds(_REM_OFF, _REM)],
                        rem_sp)
                pltpu.sync_copy(tails[slot].at[pl.ds(trow, 8), :], tail_v)
                def chunk_body(m, _):
                    for p in range(2):
                        k = m * 2 + p
                        off = pl.multiple_of(k * _CHUNK, 128)

                        @pl.when(sid == lax.rem(k, 16))
                        def _():
                            pltpu.sync_copy(
                                tables[slot].at[pl.ds(trow, 8),
                                                pl.ds(off, _CHUNK)],
                                chunk_bufs[p])
                        plsc.subcore_barrier()
                        pltpu.sync_copy(
                            chunk_bufs[p].at[c],
                            row_v.at[pl.ds(off, _CHUNK)])
                    return ()

                lax.fori_loop(0, _NCHUNK // 2, chunk_body, ())
                plsc.subcore_barrier()
                pltpu.sync_copy(rem_sp.at[c],
                                row_v.at[pl.ds(_REM_OFF, _REM)])
                for j in range(_TAIL // 16):
                    row_v[pl.ds(_TAIL_OFF + j * 16, 16)] = (
                        tail_v[c, pl.ds(j * 16, 16)])
                gather_half(fpos)
                write_band(obase)
                if dual is not None:
                    gather_half(dual)
                    write_band(dbase)

    return sc_gather


_sc_gather = _make_sc_gather()


def kernel(inputs, W_highway, W_length, W_radian, W_lon, W_lat, W_lanes,
           W_c_centrality, W_b_centrality, W_h_centrality, W_degree,
           W_cultural, W_education, W_food, W_health, W_service,
           W_transportation):
    idx = jnp.transpose(inputs[:, 2:20].astype(jnp.int32)).reshape(-1)
    tables = (W_highway, W_length, W_radian, W_lon, W_lat, W_lanes,
              W_c_centrality, W_b_centrality, W_h_centrality, W_degree,
              W_cultural, W_education, W_food, W_health, W_service,
              W_transportation)
    tt = tuple(t.T for t in tables)
    tails = tuple(t[:, _TAIL_OFF:] for t in tt)
    out_t = _sc_gather(idx, *tt, *tails)
    return jnp.transpose(out_t)


# speedup vs baseline: 1.7471x; 1.0126x over previous
import functools

import jax
import jax.numpy as jnp
from jax import lax
from jax.experimental import pallas as pl
from jax.experimental.pallas import tpu as pltpu
from jax.experimental.pallas import tpu_sc as plsc

B = 16384
V = 100000
OUT_D = 304
_HALF = B // 2
_GVECS = _HALF // 16
_CHUNK = 4096
_NCHUNK = 24
_REM_OFF = _NCHUNK * _CHUNK
_REM = 99968 - _REM_OFF
_TAIL_OFF, _TAIL = 99968, V - 99968

_FEATS = [(0, 16), (1, 16), (2, 16), (3, 32), (4, 32), (3, 32), (4, 32),
          (5, 16), (6, 16), (7, 16), (8, 16), (9, 16), (10, 8), (11, 8),
          (12, 8), (13, 8), (14, 8), (15, 8)]
_OFFS = []
_acc = 0
for _, _w in _FEATS:
    _OFFS.append(_acc)
    _acc += _w
assert _acc == OUT_D

_BANDS = []
for _f, (_slot, _w) in enumerate(_FEATS):
    if _f in (5, 6):
        continue
    for _t in range(_w // 8):
        dual = _f + 2 if _f in (3, 4) else None
        _BANDS.append((
            _slot, 8 * _t, _f, _OFFS[_f] + 8 * _t,
            dual, (_OFFS[_f + 2] + 8 * _t) if dual is not None else 0,
        ))
assert len(_BANDS) == 30

_DUALS = [b for b in _BANDS if b[4] is not None]
_PLAIN = [b for b in _BANDS if b[4] is None]
_PAIRS = ([(_DUALS[2 * i], _DUALS[2 * i + 1]) for i in range(4)]
          + [(_PLAIN[2 * i], _PLAIN[2 * i + 1]) for i in range(11)])
assert len(_PAIRS) == 15


def _make_sc_gather():
    mesh = plsc.VectorSubcoreMesh(core_axis_name="c", subcore_axis_name="s")
    scratch = [
        pltpu.VMEM_SHARED((8, _CHUNK), jnp.float32),
        pltpu.VMEM_SHARED((8, _CHUNK), jnp.float32),
        pltpu.VMEM_SHARED((8, _CHUNK), jnp.float32),
        pltpu.VMEM_SHARED((8, _CHUNK), jnp.float32),
        pltpu.VMEM_SHARED((8, _HALF), jnp.float32),
        pltpu.VMEM_SHARED((8, _REM), jnp.float32),
        pltpu.VMEM((V,), jnp.float32),
        pltpu.VMEM((8, _TAIL), jnp.float32),
        pltpu.VMEM((_HALF,), jnp.int32),
        pltpu.VMEM((_HALF,), jnp.float32),
        pltpu.SemaphoreType.DMA,
    ]

    @functools.partial(
        pl.kernel,
        out_type=jax.ShapeDtypeStruct((OUT_D, B), jnp.float32),
        mesh=mesh,
        scratch_types=scratch,
        compiler_params=pltpu.CompilerParams(needs_layout_passes=False),
    )
    def sc_gather(idx_hbm, *rest):
        tables = rest[:16]
        tails = rest[16:32]
        out_hbm = rest[32]
        (cxa, cxb, cya, cyb, out_sp, rem_sp, row_v, tail_v, idx_v,
         g_v, sem) = rest[33:44]
        bufs = ((cxa, cxb), (cya, cyb))

        cid = lax.axis_index("c")
        sid = lax.axis_index("s")
        c = lax.rem(sid, 8)
        in_y = sid // 8

        def gather_half(fpos_s, hh):
            off = pl.multiple_of(fpos_s * B + hh * _HALF, 8)
            pltpu.sync_copy(idx_hbm.at[pl.ds(off, _HALF)], idx_v)

            def body(j, _):
                iv = idx_v[pl.ds(j * 16, 16)]
                g_v[pl.ds(j * 16, 16)] = plsc.load_gather(row_v, [iv])
                return ()

            lax.fori_loop(0, _GVECS, body, (), unroll=8)

        def write_half(out_base, hh, group):
            @pl.when(in_y == group)
            def _():
                pltpu.sync_copy(g_v, out_sp.at[c])
            plsc.subcore_barrier()
            cw = _HALF // 16
            co = pl.multiple_of(sid * cw, 128)
            pltpu.sync_copy(
                out_sp.at[:, pl.ds(co, cw)],
                out_hbm.at[pl.ds(out_base, 8), pl.ds(hh * _HALF + co, cw)])
            plsc.subcore_barrier()

        for i, (bx, by) in enumerate(_PAIRS):
            (slot_x, trow_x, fpos_x, obase_x, dual_x, dbase_x) = bx
            (slot_y, trow_y, fpos_y, obase_y, dual_y, dbase_y) = by

            @pl.when(cid == (i % 2))
            def _():
                @pl.when(sid == 1)
                def _():
                    pltpu.sync_copy(
                        tables[slot_x].at[pl.ds(trow_x, 8),
                                          pl.ds(_REM_OFF, _REM)], rem_sp)

                @pl.when(in_y == 0)
                def _():
                    pltpu.sync_copy(
                        tails[slot_x].at[pl.ds(trow_x, 8), :], tail_v)

                @pl.when(in_y == 1)
                def _():
                    pltpu.sync_copy(
                        tails[slot_y].at[pl.ds(trow_y, 8), :], tail_v)

                def chunk_body(m, _):
                    for p in range(2):
                        k = m * 2 + p
                        off = pl.multiple_of(k * _CHUNK, 128)

                        @pl.when(sid == 0)
                        def _():
                            pltpu.sync_copy(
                                tables[slot_x].at[pl.ds(trow_x, 8),
                                                  pl.ds(off, _CHUNK)],
                                bufs[0][p])

                        @pl.when(sid == 8)
                        def _():
                            pltpu.sync_copy(
                                tables[slot_y].at[pl.ds(trow_y, 8),
                                                  pl.ds(off, _CHUNK)],
                                bufs[1][p])
                        plsc.subcore_barrier()

                        @pl.when(in_y == 0)
                        def _():
                            pltpu.sync_copy(bufs[0][p].at[c],
                                            row_v.at[pl.ds(off, _CHUNK)])

                        @pl.when(in_y == 1)
                        def _():
                            pltpu.sync_copy(bufs[1][p].at[c],
                                            row_v.at[pl.ds(off, _CHUNK)])
                    return ()

                lax.fori_loop(0, _NCHUNK // 2, chunk_body, ())
                plsc.subcore_barrier()

                @pl.when(in_y == 0)
                def _():
                    pltpu.sync_copy(rem_sp.at[c],
                                    row_v.at[pl.ds(_REM_OFF, _REM)])
                plsc.subcore_barrier()

                @pl.when(sid == 9)
                def _():
                    pltpu.sync_copy(
                        tables[slot_y].at[pl.ds(trow_y, 8),
                                          pl.ds(_REM_OFF, _REM)], rem_sp)
                plsc.subcore_barrier()

                @pl.when(in_y == 1)
                def _():
                    pltpu.sync_copy(rem_sp.at[c],
                                    row_v.at[pl.ds(_REM_OFF, _REM)])

                for j in range(_TAIL // 16):
                    row_v[pl.ds(_TAIL_OFF + j * 16, 16)] = (
                        tail_v[c, pl.ds(j * 16, 16)])

                fpos_s = jnp.where(in_y == 0, fpos_x, fpos_y)
                for hh in range(2):
                    gather_half(fpos_s, hh)
                    write_half(obase_x, hh, 0)
                    write_half(obase_y, hh, 1)
                if dual_x is not None:
                    dpos_s = jnp.where(in_y == 0, dual_x, dual_y)
                    for hh in range(2):
                        gather_half(dpos_s, hh)
                        write_half(dbase_x, hh, 0)
                        write_half(dbase_y, hh, 1)

    return sc_gather


_sc_gather = _make_sc_gather()


def kernel(inputs, W_highway, W_length, W_radian, W_lon, W_lat, W_lanes,
           W_c_centrality, W_b_centrality, W_h_centrality, W_degree,
           W_cultural, W_education, W_food, W_health, W_service,
           W_transportation):
    idx = jnp.transpose(inputs[:, 2:20].astype(jnp.int32)).reshape(-1)
    tables = (W_highway, W_length, W_radian, W_lon, W_lat, W_lanes,
              W_c_centrality, W_b_centrality, W_h_centrality, W_degree,
              W_cultural, W_education, W_food, W_health, W_service,
              W_transportation)
    tt = tuple(t.T for t in tables)
    tails = tuple(t[:, _TAIL_OFF:] for t in tt)
    out_t = _sc_gather(idx, *tt, *tails)
    return jnp.transpose(out_t)


# speedup vs baseline: 1.9225x; 1.1004x over previous
import functools

import jax
import jax.numpy as jnp
from jax import lax
from jax.experimental import pallas as pl
from jax.experimental.pallas import tpu as pltpu
from jax.experimental.pallas import tpu_sc as plsc

B = 16384
V = 100000
OUT_D = 304
_HALF = B // 2
_GVECS = _HALF // 16
_CHUNK = 4096
_NCHUNK = 24
_REM_OFF = _NCHUNK * _CHUNK
_REM = 99968 - _REM_OFF
_TAIL_OFF, _TAIL = 99968, V - 99968

_FEATS = [(0, 16), (1, 16), (2, 16), (3, 32), (4, 32), (3, 32), (4, 32),
          (5, 16), (6, 16), (7, 16), (8, 16), (9, 16), (10, 8), (11, 8),
          (12, 8), (13, 8), (14, 8), (15, 8)]
_OFFS = []
_acc = 0
for _, _w in _FEATS:
    _OFFS.append(_acc)
    _acc += _w
assert _acc == OUT_D

_BANDS = []
for _f, (_slot, _w) in enumerate(_FEATS):
    if _f in (5, 6):
        continue
    for _t in range(_w // 8):
        dual = _f + 2 if _f in (3, 4) else None
        _BANDS.append((
            _slot, 8 * _t, _f, _OFFS[_f] + 8 * _t,
            dual, (_OFFS[_f + 2] + 8 * _t) if dual is not None else 0,
        ))
assert len(_BANDS) == 30

_DUALS = [b for b in _BANDS if b[4] is not None]
_PLAIN = [b for b in _BANDS if b[4] is None]
_PAIRS = ([(_DUALS[2 * i], _DUALS[2 * i + 1]) for i in range(4)]
          + [(_PLAIN[2 * i], _PLAIN[2 * i + 1]) for i in range(11)])
assert len(_PAIRS) == 15


def _make_sc_gather():
    mesh = plsc.VectorSubcoreMesh(core_axis_name="c", subcore_axis_name="s")
    scratch = [
        pltpu.VMEM_SHARED((8, _CHUNK), jnp.float32),
        pltpu.VMEM_SHARED((8, _CHUNK), jnp.float32),
        pltpu.VMEM_SHARED((8, _CHUNK), jnp.float32),
        pltpu.VMEM_SHARED((8, _CHUNK), jnp.float32),
        pltpu.VMEM_SHARED((8, _HALF), jnp.float32),
        pltpu.VMEM_SHARED((8, _REM), jnp.float32),
        pltpu.VMEM((V,), jnp.float32),
        pltpu.VMEM((8, _TAIL), jnp.float32),
        pltpu.VMEM((_HALF,), jnp.int32),
        pltpu.VMEM((_HALF,), jnp.float32),
        pltpu.SemaphoreType.DMA,
        pltpu.SemaphoreType.DMA,
    ]

    @functools.partial(
        pl.kernel,
        out_type=jax.ShapeDtypeStruct((OUT_D, B), jnp.float32),
        mesh=mesh,
        scratch_types=scratch,
        compiler_params=pltpu.CompilerParams(needs_layout_passes=False),
    )
    def sc_gather(idx_hbm, *rest):
        tables = rest[:16]
        tails = rest[16:32]
        out_hbm = rest[32]
        (cxa, cxb, cya, cyb, out_sp, rem_sp, row_v, tail_v, idx_v,
         g_v, sem_x, sem_y) = rest[33:45]
        bufs = ((cxa, cxb), (cya, cyb))

        cid = lax.axis_index("c")
        sid = lax.axis_index("s")
        c = lax.rem(sid, 8)
        in_y = sid // 8

        def gather_half(fpos_s, hh):
            off = pl.multiple_of(fpos_s * B + hh * _HALF, 8)
            pltpu.sync_copy(idx_hbm.at[pl.ds(off, _HALF)], idx_v)

            def body(j, _):
                iv = idx_v[pl.ds(j * 16, 16)]
                g_v[pl.ds(j * 16, 16)] = plsc.load_gather(row_v, [iv])
                return ()

            lax.fori_loop(0, _GVECS, body, (), unroll=8)

        def write_half(out_base, hh, group):
            @pl.when(in_y == group)
            def _():
                pltpu.sync_copy(g_v, out_sp.at[c])
            plsc.subcore_barrier()
            cw = _HALF // 16
            co = pl.multiple_of(sid * cw, 128)
            pltpu.sync_copy(
                out_sp.at[:, pl.ds(co, cw)],
                out_hbm.at[pl.ds(out_base, 8), pl.ds(hh * _HALF + co, cw)])
            plsc.subcore_barrier()

        for i, (bx, by) in enumerate(_PAIRS):
            (slot_x, trow_x, fpos_x, obase_x, dual_x, dbase_x) = bx
            (slot_y, trow_y, fpos_y, obase_y, dual_y, dbase_y) = by

            @pl.when(cid == (i % 2))
            def _():
                @pl.when(sid == 1)
                def _():
                    pltpu.sync_copy(
                        tables[slot_x].at[pl.ds(trow_x, 8),
                                          pl.ds(_REM_OFF, _REM)], rem_sp)

                @pl.when(in_y == 0)
                def _():
                    pltpu.sync_copy(
                        tails[slot_x].at[pl.ds(trow_x, 8), :], tail_v)

                @pl.when(in_y == 1)
                def _():
                    pltpu.sync_copy(
                        tails[slot_y].at[pl.ds(trow_y, 8), :], tail_v)

                def issue(k, p, guard):
                    off = pl.multiple_of(k * _CHUNK, 128)

                    @pl.when(guard & (sid == 0))
                    def _():
                        pltpu.async_copy(
                            tables[slot_x].at[pl.ds(trow_x, 8),
                                              pl.ds(off, _CHUNK)],
                            bufs[0][p], sem_x)

                    @pl.when(guard & (sid == 8))
                    def _():
                        pltpu.async_copy(
                            tables[slot_y].at[pl.ds(trow_y, 8),
                                              pl.ds(off, _CHUNK)],
                            bufs[1][p], sem_y)

                issue(0, 0, sid == sid)

                def chunk_body(m, _):
                    for p in range(2):
                        k = m * 2 + p

                        @pl.when(sid == 0)
                        def _():
                            pltpu.make_async_copy(
                                tables[slot_x].at[pl.ds(trow_x, 8),
                                                  pl.ds(0, _CHUNK)],
                                bufs[0][p], sem_x).wait()

                        @pl.when(sid == 8)
                        def _():
                            pltpu.make_async_copy(
                                tables[slot_y].at[pl.ds(trow_y, 8),
                                                  pl.ds(0, _CHUNK)],
                                bufs[1][p], sem_y).wait()
                        plsc.subcore_barrier()
                        issue(k + 1, (p + 1) % 2, k + 1 < _NCHUNK)
                        off = pl.multiple_of(k * _CHUNK, 128)

                        @pl.when(in_y == 0)
                        def _():
                            pltpu.sync_copy(bufs[0][p].at[c],
                                            row_v.at[pl.ds(off, _CHUNK)])

                        @pl.when(in_y == 1)
                        def _():
                            pltpu.sync_copy(bufs[1][p].at[c],
                                            row_v.at[pl.ds(off, _CHUNK)])
                    return ()

                lax.fori_loop(0, _NCHUNK // 2, chunk_body, ())
                plsc.subcore_barrier()

                @pl.when(in_y == 0)
                def _():
                    pltpu.sync_copy(rem_sp.at[c],
                                    row_v.at[pl.ds(_REM_OFF, _REM)])
                plsc.subcore_barrier()

                @pl.when(sid == 9)
                def _():
                    pltpu.sync_copy(
                        tables[slot_y].at[pl.ds(trow_y, 8),
                                          pl.ds(_REM_OFF, _REM)], rem_sp)
                plsc.subcore_barrier()

                @pl.when(in_y == 1)
                def _():
                    pltpu.sync_copy(rem_sp.at[c],
                                    row_v.at[pl.ds(_REM_OFF, _REM)])

                for j in range(_TAIL // 16):
                    row_v[pl.ds(_TAIL_OFF + j * 16, 16)] = (
                        tail_v[c, pl.ds(j * 16, 16)])

                fpos_s = jnp.where(in_y == 0, fpos_x, fpos_y)
                for hh in range(2):
                    gather_half(fpos_s, hh)
                    write_half(obase_x, hh, 0)
                    write_half(obase_y, hh, 1)
                if dual_x is not None:
                    dpos_s = jnp.where(in_y == 0, dual_x, dual_y)
                    for hh in range(2):
                        gather_half(dpos_s, hh)
                        write_half(dbase_x, hh, 0)
                        write_half(dbase_y, hh, 1)

    return sc_gather


_sc_gather = _make_sc_gather()


def kernel(inputs, W_highway, W_length, W_radian, W_lon, W_lat, W_lanes,
           W_c_centrality, W_b_centrality, W_h_centrality, W_degree,
           W_cultural, W_education, W_food, W_health, W_service,
           W_transportation):
    idx = jnp.transpose(inputs[:, 2:20].astype(jnp.int32)).reshape(-1)
    tables = (W_highway, W_length, W_radian, W_lon, W_lat, W_lanes,
              W_c_centrality, W_b_centrality, W_h_centrality, W_degree,
              W_cultural, W_education, W_food, W_health, W_service,
              W_transportation)
    tt = tuple(t.T for t in tables)
    tails = tuple(t[:, _TAIL_OFF:] for t in tt)
    out_t = _sc_gather(idx, *tt, *tails)
    return jnp.transpose(out_t)
